# baseline (speedup 1.0000x reference)
import functools

import jax
import jax.numpy as jnp
from jax import lax
from jax.experimental import pallas as pl
from jax.experimental.pallas import tpu as pltpu
from jax.experimental.pallas import tpu_sc as plsc

NC = 2
NS = 16
NW = NC * NS
K = 128
SUP = 16
NSLOT = 8
LOOK = 4
L = 16

def _sc_agg_kernel(n_pad, n_chunks):
  base = n_chunks // NW
  rem = n_chunks % NW
  nsup0 = base // SUP
  rpt = n_pad // NS

  mesh = plsc.VectorSubcoreMesh(
      core_axis_name="c", subcore_axis_name="s",
      num_cores=NC, num_subcores=NS)

  @functools.partial(
      pl.kernel,
      out_type=(jax.ShapeDtypeStruct((NC, n_pad, 8), jnp.float32),
                jax.ShapeDtypeStruct((NC, n_pad), jnp.float32)),
      mesh=mesh,
      scratch_types=[
          pltpu.VMEM((SUP, K), jnp.int32),
          pltpu.VMEM((SUP, K), jnp.int32),
          pltpu.VMEM((K,), jnp.float32),
          [pltpu.VMEM((K, 8), jnp.float32) for _ in range(NSLOT)],
          pltpu.VMEM_SHARED((n_pad, 8), jnp.float32),
          pltpu.VMEM_SHARED((n_pad, 8), jnp.float32),
          pltpu.VMEM_SHARED((n_pad,), jnp.float32),
          pltpu.SemaphoreType.DMA,
          [pltpu.SemaphoreType.DMA for _ in range(NSLOT)],
          [pltpu.SemaphoreType.DMA for _ in range(NSLOT)],
          [pltpu.SemaphoreType.DMA for _ in range(NSLOT)],
      ],
      compiler_params=pltpu.CompilerParams(use_tc_tiling_on_sc=False),
  )
  def sc_agg(mu_hbm, row_hbm, col_hbm, zeros8_hbm, zeros1_hbm, ones_hbm,
             out_hbm, deg_hbm, rowb, colb, ones_v, vals, mu_sh, agg_sh,
             deg_sh, isem, gsems, ssems, dsems):
    c = lax.axis_index("c")
    s = lax.axis_index("s")
    w = c * NS + s
    start = w * base + jnp.minimum(w, rem)
    cpw = base + (w < rem).astype(jnp.int32)

    pltpu.sync_copy(zeros8_hbm.at[pl.ds(s * rpt, rpt)],
                    agg_sh.at[pl.ds(s * rpt, rpt)])
    pltpu.sync_copy(zeros1_hbm.at[pl.ds(s * rpt, rpt)],
                    deg_sh.at[pl.ds(s * rpt, rpt)])
    pltpu.sync_copy(mu_hbm.at[pl.ds(s * rpt, rpt)],
                    mu_sh.at[pl.ds(s * rpt, rpt)])
    pltpu.sync_copy(ones_hbm, ones_v)
    plsc.subcore_barrier()

    def body(sup, carry):
      chunk0 = start + sup * SUP
      ic1 = pltpu.async_copy(row_hbm.at[pl.ds(chunk0, SUP)], rowb, isem)
      ic2 = pltpu.async_copy(col_hbm.at[pl.ds(chunk0, SUP)], colb, isem)
      ic1.wait()
      ic2.wait()
      gcps = [None] * SUP
      scps = [None] * SUP
      dcps = [None] * SUP
      for j in range(LOOK):
        gcps[j] = pltpu.async_copy(
            mu_sh.at[colb.at[j]], vals[j % NSLOT], gsems[j % NSLOT])
      for j in range(SUP):
        nxt = j + LOOK
        if nxt < SUP:
          if nxt - NSLOT >= 0:
            scps[nxt - NSLOT].wait()
            dcps[nxt - NSLOT].wait()
          gcps[nxt] = pltpu.async_copy(
              mu_sh.at[colb.at[nxt]], vals[nxt % NSLOT], gsems[nxt % NSLOT])
        gcps[j].wait()
        scps[j] = pltpu.async_copy(
            vals[j % NSLOT], agg_sh.at[rowb.at[j]], ssems[j % NSLOT],
            add=True)
        dcps[j] = pltpu.async_copy(
            ones_v, deg_sh.at[rowb.at[j]], dsems[j % NSLOT], add=True)
      for j in range(SUP - NSLOT, SUP):
        scps[j].wait()
        dcps[j].wait()
      return carry

    nsup = cpw // SUP
    lax.fori_loop(0, nsup, body, 0)

    tail = cpw - nsup * SUP
    tb0 = start + cpw - SUP
    tc1 = pltpu.async_copy(row_hbm.at[pl.ds(tb0, SUP)], rowb, isem)
    tc2 = pltpu.async_copy(col_hbm.at[pl.ds(tb0, SUP)], colb, isem)
    tc1.wait()
    tc2.wait()

    def tail_body(j, carry):
      pltpu.async_copy(mu_sh.at[colb.at[j]], vals[0], gsems[0]).wait()
      pltpu.async_copy(
          vals[0], agg_sh.at[rowb.at[j]], ssems[0], add=True).wait()
      pltpu.async_copy(
          ones_v, deg_sh.at[rowb.at[j]], dsems[0], add=True).wait()
      return carry

    lax.fori_loop(SUP - tail, SUP, tail_body, 0)
    plsc.subcore_barrier()

    pltpu.sync_copy(agg_sh.at[pl.ds(s * rpt, rpt)],
                    out_hbm.at[c, pl.ds(s * rpt, rpt)])
    pltpu.sync_copy(deg_sh.at[pl.ds(s * rpt, rpt)],
                    deg_hbm.at[c, pl.ds(s * rpt, rpt)])

  return sc_agg


def _erf(x):
  z = jnp.abs(x)
  t = 1.0 / (1.0 + 0.3275911 * z)
  poly = t * (0.254829592 + t * (-0.284496736 + t * (1.421413741
             + t * (-1.453152027 + t * 1.061405429))))
  r = 1.0 - poly * jnp.exp(-z * z)
  return jnp.where(x < 0, -r, r)


def _mlp_block_kernel(part_ref, deg_ref, w1_ref, b1_ref, w2_ref, b2_ref,
                      out_ref):
  x = part_ref[0] + part_ref[1]
  deg = jnp.maximum(deg_ref[0] + deg_ref[1], 1.0)
  s = x / deg
  w1 = w1_ref[...]
  b1 = b1_ref[...]
  w2 = w2_ref[...]
  b2 = b2_ref[0, 0]
  cols = []
  for bb in range(8):
    h = s[:, bb:bb + 1] * w1 + b1
    g = 0.5 * h * (1.0 + _erf(h * 0.7071067811865476))
    g = g.astype(jnp.bfloat16).astype(jnp.float32)
    yb = jnp.sum(g * w2, axis=1, keepdims=True) + b2
    cols.append(yb)
  out_ref[...] = jnp.concatenate(cols, axis=1)


def _mlp_call(partial, degp3, w1f, b1f, w2f, b2f, hdim, n_pad):
  nb = 256
  grid = (n_pad + nb - 1) // nb
  return pl.pallas_call(
      _mlp_block_kernel,
      grid=(grid,),
      in_specs=[
          pl.BlockSpec((NC, nb, 8), lambda i: (0, i, 0)),
          pl.BlockSpec((NC, nb, 1), lambda i: (0, i, 0)),
          pl.BlockSpec((1, hdim), lambda i: (0, 0)),
          pl.BlockSpec((1, hdim), lambda i: (0, 0)),
          pl.BlockSpec((1, hdim), lambda i: (0, 0)),
          pl.BlockSpec((1, 1), lambda i: (0, 0)),
      ],
      out_specs=pl.BlockSpec((nb, 8), lambda i: (i, 0)),
      out_shape=jax.ShapeDtypeStruct((n_pad, 8), jnp.float32),
  )(partial, degp3, w1f, b1f, w2f, b2f)


def kernel(mu, edge_index, W1, b1, W2, b2):
  bsz, n = mu.shape
  e = edge_index.shape[1]
  hdim = W1.shape[0]

  n_pad = (n // (L * NW) + 1) * L * NW
  mu_t = jnp.concatenate(
      [mu.T, jnp.zeros((n_pad - n, bsz), jnp.float32)])

  if e % K == 0:
    row_flat, col_flat = edge_index[0], edge_index[1]
    e_pad = e
  else:
    pad = K - e % K
    row_flat = jnp.concatenate([edge_index[0], jnp.full((pad,), n, jnp.int32)])
    col_flat = jnp.concatenate([edge_index[1], jnp.zeros((pad,), jnp.int32)])
    e_pad = e + pad
  n_chunks = e_pad // K
  row3 = row_flat.reshape(n_chunks, K)
  col3 = col_flat.reshape(n_chunks, K)
  zeros8 = jnp.zeros((n_pad, 8), jnp.float32)
  zeros1 = jnp.zeros((n_pad,), jnp.float32)
  ones_k = jnp.ones((K,), jnp.float32)

  partial, degp = _sc_agg_kernel(n_pad, n_chunks)(
      mu_t, row3, col3, zeros8, zeros1, ones_k)

  w1f = W1.reshape(1, hdim)
  b1f = b1.reshape(1, hdim)
  w2f = W2.astype(jnp.bfloat16).astype(jnp.float32).reshape(1, hdim)
  b2f = b2.reshape(1, 1)
  degp3 = degp.reshape(NC, n_pad, 1)
  yt = _mlp_call(partial, degp3, w1f, b1f, w2f, b2f, hdim, n_pad)

  return yt[:n].T


# speedup vs baseline: 1.2037x; 1.2037x over previous
import functools

import jax
import jax.numpy as jnp
from jax import lax
from jax.experimental import pallas as pl
from jax.experimental.pallas import tpu as pltpu
from jax.experimental.pallas import tpu_sc as plsc

NC = 2
NS = 16
NW = NC * NS
K = 128
SUP = 16
NSLOT = 8
LOOK = 4


def _sc_agg_kernel(n_pad, n_chunks):
  cpw = n_chunks // NW
  nsup = cpw // SUP
  rpt = n_pad // NS

  mesh = plsc.VectorSubcoreMesh(
      core_axis_name="c", subcore_axis_name="s",
      num_cores=NC, num_subcores=NS)

  @functools.partial(
      pl.kernel,
      out_type=jax.ShapeDtypeStruct((NC, n_pad, 16), jnp.float32),
      mesh=mesh,
      scratch_types=[
          pltpu.VMEM((SUP, K), jnp.int32),
          pltpu.VMEM((SUP, K), jnp.int32),
          [pltpu.VMEM((K, 16), jnp.float32) for _ in range(NSLOT)],
          pltpu.VMEM_SHARED((n_pad, 16), jnp.float32),
          pltpu.SemaphoreType.DMA,
          [pltpu.SemaphoreType.DMA for _ in range(NSLOT)],
          [pltpu.SemaphoreType.DMA for _ in range(NSLOT)],
      ],
      compiler_params=pltpu.CompilerParams(use_tc_tiling_on_sc=False),
  )
  def sc_agg(mu_hbm, row_hbm, col_hbm, zeros_hbm, out_hbm,
             rowb, colb, vals, agg_sh, isem, gsems, ssems):
    c = lax.axis_index("c")
    s = lax.axis_index("s")
    w = c * NS + s

    pltpu.sync_copy(zeros_hbm.at[pl.ds(s * rpt, rpt)],
                    agg_sh.at[pl.ds(s * rpt, rpt)])
    plsc.subcore_barrier()

    def body(sup, carry):
      chunk0 = w * cpw + sup * SUP
      ic1 = pltpu.async_copy(row_hbm.at[pl.ds(chunk0, SUP)], rowb, isem)
      ic2 = pltpu.async_copy(col_hbm.at[pl.ds(chunk0, SUP)], colb, isem)
      ic1.wait()
      ic2.wait()
      gcps = [None] * SUP
      scps = [None] * SUP
      for j in range(LOOK):
        gcps[j] = pltpu.async_copy(
            mu_hbm.at[colb.at[j]], vals[j % NSLOT], gsems[j % NSLOT])
      for j in range(SUP):
        nxt = j + LOOK
        if nxt < SUP:
          if nxt - NSLOT >= 0:
            scps[nxt - NSLOT].wait()
          gcps[nxt] = pltpu.async_copy(
              mu_hbm.at[colb.at[nxt]], vals[nxt % NSLOT], gsems[nxt % NSLOT])
        gcps[j].wait()
        scps[j] = pltpu.async_copy(
            vals[j % NSLOT], agg_sh.at[rowb.at[j]], ssems[j % NSLOT],
            add=True)
      for j in range(SUP - NSLOT, SUP):
        scps[j].wait()
      return carry

    lax.fori_loop(0, nsup, body, 0)
    plsc.subcore_barrier()

    pltpu.sync_copy(agg_sh.at[pl.ds(s * rpt, rpt)],
                    out_hbm.at[c, pl.ds(s * rpt, rpt)])

  return sc_agg


def _erf(x):
  z = jnp.abs(x)
  t = 1.0 / (1.0 + 0.3275911 * z)
  poly = t * (0.254829592 + t * (-0.284496736 + t * (1.421413741
             + t * (-1.453152027 + t * 1.061405429))))
  r = 1.0 - poly * jnp.exp(-z * z)
  return jnp.where(x < 0, -r, r)


def _mlp_block_kernel(part_ref, w1_ref, b1_ref, w2_ref, b2_ref, out_ref):
  x = part_ref[0] + part_ref[1]
  deg = jnp.maximum(x[8:9, :], 1.0)
  w1 = w1_ref[...]
  b1 = b1_ref[...]
  w2 = w2_ref[...]
  b2 = b2_ref[0, 0]
  rows = []
  for bb in range(8):
    s = x[bb:bb + 1, :] / deg
    h = s * w1 + b1
    g = 0.5 * h * (1.0 + _erf(h * 0.7071067811865476))
    g = g.astype(jnp.bfloat16).astype(jnp.float32)
    yb = jnp.sum(g * w2, axis=0, keepdims=True) + b2
    rows.append(yb)
  out_ref[...] = jnp.concatenate(rows, axis=0)


def kernel(mu, edge_index, W1, b1, W2, b2):
  bsz, n = mu.shape
  e = edge_index.shape[1]
  hdim = W1.shape[0]

  mu_t = mu.T
  mu_ext = jnp.concatenate(
      [mu_t,
       jnp.ones((n, 1), jnp.float32),
       jnp.zeros((n, 16 - bsz - 1), jnp.float32)], axis=1)
  n_pad = (n // (8 * NS) + 1) * 8 * NS

  unit = NW * K * SUP
  e_pad = ((e + unit - 1) // unit) * unit
  n_chunks = e_pad // K
  pad = e_pad - e
  row3 = jnp.concatenate(
      [edge_index[0], jnp.full((pad,), n, jnp.int32)]).reshape(n_chunks, K)
  col3 = jnp.concatenate(
      [edge_index[1], jnp.zeros((pad,), jnp.int32)]).reshape(n_chunks, K)
  zeros_init = jnp.zeros((n_pad, 16), jnp.float32)

  partial = _sc_agg_kernel(n_pad, n_chunks)(mu_ext, row3, col3, zeros_init)

  part_t = jnp.transpose(partial, (0, 2, 1))
  w1f = W1.reshape(hdim, 1)
  b1f = b1.reshape(hdim, 1)
  w2f = W2.astype(jnp.bfloat16).astype(jnp.float32).reshape(hdim, 1)
  b2f = b2.reshape(1, 1)
  nb = 512
  grid = (n_pad + nb - 1) // nb
  yt = pl.pallas_call(
      _mlp_block_kernel,
      grid=(grid,),
      in_specs=[
          pl.BlockSpec((NC, 16, nb), lambda i: (0, 0, i)),
          pl.BlockSpec((hdim, 1), lambda i: (0, 0)),
          pl.BlockSpec((hdim, 1), lambda i: (0, 0)),
          pl.BlockSpec((hdim, 1), lambda i: (0, 0)),
          pl.BlockSpec((1, 1), lambda i: (0, 0)),
      ],
      out_specs=pl.BlockSpec((8, nb), lambda i: (0, i)),
      out_shape=jax.ShapeDtypeStruct((8, n_pad), jnp.float32),
  )(part_t, w1f, b1f, w2f, b2f)

  return yt[:, :n]
